# baseline (speedup 1.0000x reference)
import functools

import jax
import jax.numpy as jnp
from jax import lax
from jax.experimental import pallas as pl
from jax.experimental.pallas import tpu as pltpu
from jax.experimental.pallas import tpu_sc as plsc

D_MODEL = 128
HEIGHT = 512
CENTER = 256 * 256 - 1
BAND_START = CENTER - (HEIGHT - 1)
BAND_ROWS = 1024
NUM_WORKERS = 32
H_PER_W = HEIGHT // NUM_WORKERS
JQ = 64
NQ = HEIGHT // JQ
NBUF = 4
NTILES = H_PER_W * NQ
NGROUPS = NTILES // NBUF
LANES = 16
D_GROUPS = D_MODEL // LANES


def _sc_body(w_hbm, x_hbm, o_hbm, band_sh, x_v, b0, b1, b2, b3,
             si0, si1, si2, si3, so0, so1, so2, so3):
    cid = lax.axis_index("c")
    sid = lax.axis_index("s")
    wid = sid * 2 + cid
    h0 = wid * H_PER_W

    bufs = (b0, b1, b2, b3)
    sin = (si0, si1, si2, si3)
    sout = (so0, so1, so2, so3)

    @pl.when(sid == 0)
    def _load_band():
        pltpu.sync_copy(w_hbm.at[pl.ds(BAND_START, BAND_ROWS)], band_sh)

    pltpu.sync_copy(x_hbm, x_v)
    plsc.subcore_barrier()

    def start_in(i, b):
        t = i // NQ
        q = i - t * NQ
        g0 = (HEIGHT - 1) + q * JQ - h0 - t
        pltpu.async_copy(band_sh.at[pl.ds(g0, JQ)], bufs[b], sin[b])

    def wait_in(b):
        pltpu.make_async_copy(band_sh.at[pl.ds(0, JQ)], bufs[b], sin[b]).wait()

    def start_out(i, b):
        t = i // NQ
        q = i - t * NQ
        pltpu.async_copy(bufs[b], o_hbm.at[h0 + t, pl.ds(q * JQ, JQ)], sout[b])

    def wait_out(b):
        pltpu.make_async_copy(bufs[b], o_hbm.at[h0, pl.ds(0, JQ)], sout[b]).wait()

    def add_x(i, b):
        buf = bufs[b]
        t = i // NQ
        jq0 = (i - t * NQ) * JQ

        @plsc.parallel_loop(0, JQ // LANES)
        def _jc_body(jc):
            jbase = jq0 + jc * LANES
            xv = x_v[pl.ds(jbase, LANES)]
            for i_ in range(LANES):
                jj = jc * LANES + i_
                splat = jnp.full((LANES,), xv[i_], jnp.float32)
                for c in range(D_GROUPS):
                    plsc.addupdate(buf.at[jj, pl.ds(c * LANES, LANES)], splat)

    start_in(0, 0)
    start_in(1, 1)

    def g_body(g, carry):
        for b in range(NBUF):
            i = g * NBUF + b
            i2 = i + 2
            b2 = (b + 2) % NBUF
            if b < 2:
                @pl.when(g >= 1)
                def _drain():
                    wait_out(b2)

                start_in(i2, b2)
            else:
                @pl.when(g <= NGROUPS - 2)
                def _rearm():
                    wait_out(b2)
                    start_in(i2, b2)

            wait_in(b)
            add_x(i, b)
            start_out(i, b)
        return carry

    lax.fori_loop(0, NGROUPS, g_body, 0)
    for b in range(NBUF):
        wait_out(b)


_sc_kernel = functools.partial(
    pl.kernel,
    out_type=jax.ShapeDtypeStruct((HEIGHT, HEIGHT, D_MODEL), jnp.float32),
    mesh=plsc.VectorSubcoreMesh(core_axis_name="c", subcore_axis_name="s"),
    scratch_types=[
        pltpu.VMEM_SHARED((BAND_ROWS, D_MODEL), jnp.float32),
        pltpu.VMEM((HEIGHT,), jnp.float32),
        pltpu.VMEM((JQ, D_MODEL), jnp.float32),
        pltpu.VMEM((JQ, D_MODEL), jnp.float32),
        pltpu.VMEM((JQ, D_MODEL), jnp.float32),
        pltpu.VMEM((JQ, D_MODEL), jnp.float32),
        pltpu.SemaphoreType.DMA,
        pltpu.SemaphoreType.DMA,
        pltpu.SemaphoreType.DMA,
        pltpu.SemaphoreType.DMA,
        pltpu.SemaphoreType.DMA,
        pltpu.SemaphoreType.DMA,
        pltpu.SemaphoreType.DMA,
        pltpu.SemaphoreType.DMA,
    ],
)(_sc_body)


def kernel(x, weight):
    xr = x.reshape(HEIGHT)
    out = _sc_kernel(weight, xr)
    return out.reshape(1, HEIGHT, 1, HEIGHT, D_MODEL)


# speedup vs baseline: 1.0210x; 1.0210x over previous
import functools

import jax
import jax.numpy as jnp
from jax import lax
from jax.experimental import pallas as pl
from jax.experimental.pallas import tpu as pltpu
from jax.experimental.pallas import tpu_sc as plsc

D_MODEL = 128
HEIGHT = 512
CENTER = 256 * 256 - 1
BAND_START = CENTER - (HEIGHT - 1)
BAND_ROWS = 1024
NUM_WORKERS = 32
H_PER_W = HEIGHT // NUM_WORKERS
JQ = 128
NQ = HEIGHT // JQ
NBUF = 4
NTILES = H_PER_W * NQ
NGROUPS = NTILES // NBUF
LANES = 16
D_GROUPS = D_MODEL // LANES


def _sc_body(w_hbm, x_hbm, o_hbm, band_sh, x_v, b0, b1, b2, b3,
             si0, si1, si2, si3, so0, so1, so2, so3):
    cid = lax.axis_index("c")
    sid = lax.axis_index("s")
    wid = sid * 2 + cid
    h0 = wid * H_PER_W

    bufs = (b0, b1, b2, b3)
    sin = (si0, si1, si2, si3)
    sout = (so0, so1, so2, so3)

    stripe = BAND_ROWS // 16
    pltpu.sync_copy(
        w_hbm.at[pl.ds(BAND_START + sid * stripe, stripe)],
        band_sh.at[pl.ds(sid * stripe, stripe)],
    )
    pltpu.sync_copy(x_hbm, x_v)
    plsc.subcore_barrier()

    def start_in(i, b):
        t = i // NQ
        q = i - t * NQ
        g0 = (HEIGHT - 1) + q * JQ - h0 - t
        pltpu.async_copy(band_sh.at[pl.ds(g0, JQ)], bufs[b], sin[b])

    def wait_in(b):
        pltpu.make_async_copy(band_sh.at[pl.ds(0, JQ)], bufs[b], sin[b]).wait()

    def start_out(i, b):
        t = i // NQ
        q = i - t * NQ
        pltpu.async_copy(bufs[b], o_hbm.at[h0 + t, pl.ds(q * JQ, JQ)], sout[b])

    def wait_out(b):
        pltpu.make_async_copy(bufs[b], o_hbm.at[h0, pl.ds(0, JQ)], sout[b]).wait()

    def add_x(i, b):
        buf = bufs[b]
        t = i // NQ
        jq0 = (i - t * NQ) * JQ

        @plsc.parallel_loop(0, JQ // LANES)
        def _jc_body(jc):
            jbase = jq0 + jc * LANES
            xv = x_v[pl.ds(jbase, LANES)]
            for i_ in range(LANES):
                jj = jc * LANES + i_
                splat = jnp.full((LANES,), xv[i_], jnp.float32)
                for c in range(D_GROUPS):
                    plsc.addupdate(buf.at[jj, pl.ds(c * LANES, LANES)], splat)

    start_in(0, 0)
    start_in(1, 1)

    def g_body(g, carry):
        for b in range(NBUF):
            i = g * NBUF + b
            i2 = i + 2
            b2 = (b + 2) % NBUF
            if b < 2:
                @pl.when(g >= 1)
                def _drain():
                    wait_out(b2)

                start_in(i2, b2)
            else:
                @pl.when(g <= NGROUPS - 2)
                def _rearm():
                    wait_out(b2)
                    start_in(i2, b2)

            wait_in(b)
            add_x(i, b)
            start_out(i, b)
        return carry

    lax.fori_loop(0, NGROUPS, g_body, 0)
    for b in range(NBUF):
        wait_out(b)


_sc_kernel = functools.partial(
    pl.kernel,
    out_type=jax.ShapeDtypeStruct((HEIGHT, HEIGHT, D_MODEL), jnp.float32),
    mesh=plsc.VectorSubcoreMesh(core_axis_name="c", subcore_axis_name="s"),
    scratch_types=[
        pltpu.VMEM_SHARED((BAND_ROWS, D_MODEL), jnp.float32),
        pltpu.VMEM((HEIGHT,), jnp.float32),
        pltpu.VMEM((JQ, D_MODEL), jnp.float32),
        pltpu.VMEM((JQ, D_MODEL), jnp.float32),
        pltpu.VMEM((JQ, D_MODEL), jnp.float32),
        pltpu.VMEM((JQ, D_MODEL), jnp.float32),
        pltpu.SemaphoreType.DMA,
        pltpu.SemaphoreType.DMA,
        pltpu.SemaphoreType.DMA,
        pltpu.SemaphoreType.DMA,
        pltpu.SemaphoreType.DMA,
        pltpu.SemaphoreType.DMA,
        pltpu.SemaphoreType.DMA,
        pltpu.SemaphoreType.DMA,
    ],
)(_sc_body)


def kernel(x, weight):
    xr = x.reshape(HEIGHT)
    out = _sc_kernel(weight, xr)
    return out.reshape(1, HEIGHT, 1, HEIGHT, D_MODEL)
